# baseline (speedup 1.0000x reference)
import jax
import jax.numpy as jnp
from jax.experimental import pallas as pl
from jax.experimental.pallas import tpu as pltpu

_TOP_K = 2
_TB = 2048
_HALF = _TB // 2


def _z_copies(z_ref, zb_ref, i, sem_a, sem_b):
    return (
        pltpu.make_async_copy(
            zb_ref.at[pl.ds(0, _HALF), :],
            z_ref.at[pl.ds(i * _TB, _HALF), :], sem_a),
        pltpu.make_async_copy(
            zb_ref.at[pl.ds(_HALF, _HALF), :],
            z_ref.at[pl.ds(i * _TB + _HALF, _HALF), :], sem_b),
    )


def _routing_kernel(x_ref, gw_ref, z_ref, m_ref, zb_ref, sem_a, sem_b):
    i = pl.program_id(0)
    n = pl.num_programs(0)

    @pl.when(i == 0)
    def _init():
        zb_ref[...] = jnp.zeros_like(zb_ref)

    @pl.when(i > 0)
    def _drain_prev():
        for c in _z_copies(z_ref, zb_ref, i - 1, sem_a, sem_b):
            c.wait()

    cur = _z_copies(z_ref, zb_ref, i, sem_a, sem_b)
    for c in cur:
        c.start()

    lt = jax.lax.dot_general(
        gw_ref[...], x_ref[...],
        dimension_numbers=(((1,), (1,)), ((), ())),
        preferred_element_type=jnp.float32,
    )
    E = lt.shape[0]
    eidx = jax.lax.broadcasted_iota(jnp.int32, lt.shape, 0)
    m1 = jnp.max(lt, axis=0, keepdims=True)
    i1 = jnp.min(jnp.where(lt == m1, eidx, E), axis=0, keepdims=True)
    lt2 = jnp.where(eidx == i1, -jnp.inf, lt)
    m2 = jnp.max(lt2, axis=0, keepdims=True)
    i2 = jnp.min(jnp.where(lt2 == m2, eidx, E), axis=0, keepdims=True)
    e3 = jax.lax.broadcasted_iota(jnp.int32, m_ref.shape, 0)
    k3 = jax.lax.broadcasted_iota(jnp.int32, m_ref.shape, 1)
    sel = jnp.where(k3 == 0, i1[None], i2[None])
    m_ref[...] = (e3 == sel).astype(jnp.float32)

    @pl.when(i == n - 1)
    def _drain_last():
        for c in cur:
            c.wait()


def kernel(hidden_states, gate_w):
    b, s, h = hidden_states.shape
    t = b * s
    e = gate_w.shape[0]
    x = hidden_states.reshape(t, h)
    grid = (t // _TB,)
    z, m = pl.pallas_call(
        _routing_kernel,
        grid=grid,
        in_specs=[
            pl.BlockSpec((_TB, h), lambda i: (i, 0)),
            pl.BlockSpec((e, h), lambda i: (0, 0)),
        ],
        out_specs=[
            pl.BlockSpec(memory_space=pl.ANY),
            pl.BlockSpec((e, _TOP_K, _TB), lambda i: (0, 0, i)),
        ],
        out_shape=[
            jax.ShapeDtypeStruct((t, h), jnp.float32),
            jax.ShapeDtypeStruct((e, _TOP_K, t), jnp.float32),
        ],
        scratch_shapes=[
            pltpu.VMEM((_TB, h), jnp.float32),
            pltpu.SemaphoreType.DMA,
            pltpu.SemaphoreType.DMA,
        ],
    )(x, gate_w)
    return (z, m)


# speedup vs baseline: 1.0033x; 1.0033x over previous
import jax
import jax.numpy as jnp
from jax.experimental import pallas as pl
from jax.experimental.pallas import tpu as pltpu

_TOP_K = 2
_TB = 2048


def _routing_kernel(x_ref, gw_ref, z_ref, m_ref, zb_ref, sem):
    i = pl.program_id(0)
    n = pl.num_programs(0)

    @pl.when(i == 0)
    def _init():
        zb_ref[...] = jnp.zeros_like(zb_ref)

    @pl.when(i > 0)
    def _drain_prev():
        pltpu.make_async_copy(
            zb_ref, z_ref.at[pl.ds((i - 1) * _TB, _TB), :], sem
        ).wait()

    cur = pltpu.make_async_copy(
        zb_ref, z_ref.at[pl.ds(i * _TB, _TB), :], sem
    )
    cur.start()

    lt = jax.lax.dot_general(
        gw_ref[...], x_ref[...],
        dimension_numbers=(((1,), (1,)), ((), ())),
        preferred_element_type=jnp.float32,
    )
    E = lt.shape[0]
    eidx = jax.lax.broadcasted_iota(jnp.int32, lt.shape, 0)
    m1 = jnp.max(lt, axis=0, keepdims=True)
    i1 = jnp.min(jnp.where(lt == m1, eidx, E), axis=0, keepdims=True)
    lt2 = jnp.where(eidx == i1, -jnp.inf, lt)
    m2 = jnp.max(lt2, axis=0, keepdims=True)
    i2 = jnp.min(jnp.where(lt2 == m2, eidx, E), axis=0, keepdims=True)
    e3 = jax.lax.broadcasted_iota(jnp.int32, m_ref.shape, 0)
    k3 = jax.lax.broadcasted_iota(jnp.int32, m_ref.shape, 1)
    sel = jnp.where(k3 == 0, i1[None], i2[None])
    m_ref[...] = (e3 == sel).astype(jnp.float32)

    @pl.when(i == n - 1)
    def _drain_last():
        cur.wait()


def kernel(hidden_states, gate_w):
    b, s, h = hidden_states.shape
    t = b * s
    e = gate_w.shape[0]
    x = hidden_states.reshape(t, h)
    grid = (t // _TB,)
    z, m = pl.pallas_call(
        _routing_kernel,
        grid=grid,
        in_specs=[
            pl.BlockSpec((_TB, h), lambda i: (i, 0)),
            pl.BlockSpec((e, h), lambda i: (0, 0)),
        ],
        out_specs=[
            pl.BlockSpec(memory_space=pl.ANY),
            pl.BlockSpec((e, _TOP_K, _TB), lambda i: (0, 0, i)),
        ],
        out_shape=[
            jax.ShapeDtypeStruct((t, h), jnp.float32),
            jax.ShapeDtypeStruct((e, _TOP_K, t), jnp.float32),
        ],
        scratch_shapes=[
            pltpu.VMEM((_TB, h), jnp.float32),
            pltpu.SemaphoreType.DMA,
        ],
    )(x, gate_w)
    return (z, m)
